# baseline (speedup 1.0000x reference)
import functools

import jax
import jax.numpy as jnp
from jax import lax
from jax.experimental import pallas as pl
from jax.experimental.pallas import tpu as pltpu
from jax.experimental.pallas import tpu_sc as plsc

N = 10000
NP = 10240
E = 160000
DIM = 256
DIM_DST = 32
NBASIS = 8
ZDIM = 16
CUTOFF = 5.0
BETA = (NBASIS / CUTOFF) ** 2
CENTERS = [float(i) * CUTOFF / (NBASIS - 1) for i in range(NBASIS)]

NC, NS, L = 2, 16, 16
ZCAT = ZDIM + DIM_DST

EPT = E // NS
CK = 80
NCH = EPT // CK
GPC = CK // L
NB_HALF = NBASIS // NC
FH = NB_HALF * DIM_DST
RPT = NP // NS

_MESH = dict(core_axis_name="c", subcore_axis_name="s", num_cores=NC,
             num_subcores=NS)


def _sc_gather_body(table_hbm, idx_hbm, out_hbm, idxv, rows, sem):
    c = lax.axis_index("c")
    s = lax.axis_index("s")
    wid = s * NC + c
    base = wid * (NP // (NC * NS))
    for j in range(5):
        off = base + j * 64
        pltpu.sync_copy(idx_hbm.at[pl.ds(off, 64)], idxv)
        pltpu.async_copy(table_hbm.at[idxv], rows, sem).wait()
        pltpu.sync_copy(rows, out_hbm.at[pl.ds(off, 64)])


def _sc_gather(table, idx):
    k = pl.kernel(
        _sc_gather_body,
        out_type=jax.ShapeDtypeStruct((NP, ZCAT), jnp.float32),
        mesh=plsc.VectorSubcoreMesh(**_MESH),
        scratch_types=[
            pltpu.VMEM((64,), jnp.int32),
            pltpu.VMEM((64, ZCAT), jnp.float32),
            pltpu.SemaphoreType.DMA,
        ],
        compiler_params=pltpu.CompilerParams(use_tc_tiling_on_sc=False),
    )
    return k(table, idx)


def _sc_edge_body(s_hbm, src_hbm, dst_hbm, d_hbm, sw_hbm, out_hbm,
                  srcv, dstv, dv, swv, xij, lij, acc, sem):
    c = lax.axis_index("c")
    sid = lax.axis_index("s")
    zero16 = jnp.zeros((L,), jnp.float32)

    def zrow(r, _):
        for j in range(FH // L):
            lij[r, pl.ds(j * L, L)] = zero16
        return 0
    lax.fori_loop(0, CK, zrow, 0)
    for j in range(RPT // CK):
        pltpu.sync_copy(lij, acc.at[pl.ds(sid * RPT + j * CK, CK)])
    plsc.subcore_barrier()

    def chunk(i, _):
        off = pl.multiple_of(sid * EPT + i * CK, CK)
        pltpu.sync_copy(src_hbm.at[pl.ds(off, CK)], srcv)
        pltpu.sync_copy(dst_hbm.at[pl.ds(off, CK)], dstv)
        pltpu.sync_copy(d_hbm.at[pl.ds(off, CK)], dv)
        pltpu.sync_copy(sw_hbm.at[pl.ds(off, CK)], swv)
        pltpu.async_copy(s_hbm.at[dstv], xij, sem).wait()
        for g in range(GPC):
            dval = dv[pl.ds(g * L, L)]
            swval = swv[pl.ds(g * L, L)]
            rowv = g * L + lax.iota(jnp.int32, L)
            rbs = []
            for b in range(NB_HALF):
                cb = jnp.where(c == 0, CENTERS[b], CENTERS[b + NB_HALF])
                diff = dval - cb
                rbs.append(jnp.exp((-BETA) * diff * diff) * swval)
            for col in range(DIM_DST):
                colv = jnp.full((L,), col, jnp.int32)
                xcol = plsc.load_gather(xij, [rowv, colv])
                for b in range(NB_HALF):
                    fcol = jnp.full((L,), b * DIM_DST + col, jnp.int32)
                    plsc.store_scatter(lij, [rowv, fcol], rbs[b] * xcol)
        pltpu.sync_copy(lij, acc.at[srcv], add=True)
        return 0

    lax.fori_loop(0, NCH, chunk, 0)
    plsc.subcore_barrier()
    r0 = sid * RPT
    pltpu.sync_copy(acc.at[pl.ds(r0, RPT)], out_hbm.at[c, pl.ds(r0, RPT)])


def _sc_edge(s_nodes, src, dst, d, sw):
    k = pl.kernel(
        _sc_edge_body,
        out_type=jax.ShapeDtypeStruct((NC, NP, FH), jnp.float32),
        mesh=plsc.VectorSubcoreMesh(**_MESH),
        scratch_types=[
            pltpu.VMEM((CK,), jnp.int32),
            pltpu.VMEM((CK,), jnp.int32),
            pltpu.VMEM((CK,), jnp.float32),
            pltpu.VMEM((CK,), jnp.float32),
            pltpu.VMEM((CK, DIM_DST), jnp.float32),
            pltpu.VMEM((CK, FH), jnp.float32),
            pltpu.VMEM_SHARED((NP, FH), jnp.float32),
            pltpu.SemaphoreType.DMA,
        ],
        compiler_params=pltpu.CompilerParams(use_tc_tiling_on_sc=False,
                                             needs_layout_passes=False),
    )
    return k(s_nodes, src, dst, d, sw)


TBN = 2048


def _tc_mix_body(x_ref, m0_ref, m1_ref, wx_ref, w0_ref, w1_ref, bm_ref,
                 wsn_ref, xi_ref, s_ref):
    h = jnp.dot(x_ref[...], wx_ref[...], preferred_element_type=jnp.float32)
    h = h + jnp.dot(m0_ref[...], w0_ref[...], preferred_element_type=jnp.float32)
    h = h + jnp.dot(m1_ref[...], w1_ref[...], preferred_element_type=jnp.float32)
    h = h + bm_ref[...]
    xi = h * jnp.exp(jnp.log(1.0 + h * h) * (-1.0 / 3.0))
    xi_ref[...] = xi
    s_ref[...] = jnp.dot(xi, wsn_ref[...], preferred_element_type=jnp.float32)


def _tc_mix(x, m0, m1, wx, w0, w1, bm, wsn):
    kx = x.shape[1]
    return pl.pallas_call(
        _tc_mix_body,
        grid=(NP // TBN,),
        in_specs=[
            pl.BlockSpec((TBN, kx), lambda i: (i, 0)),
            pl.BlockSpec((TBN, FH), lambda i: (i, 0)),
            pl.BlockSpec((TBN, FH), lambda i: (i, 0)),
            pl.BlockSpec((kx, DIM), lambda i: (0, 0)),
            pl.BlockSpec((FH, DIM), lambda i: (0, 0)),
            pl.BlockSpec((FH, DIM), lambda i: (0, 0)),
            pl.BlockSpec((1, DIM), lambda i: (0, 0)),
            pl.BlockSpec((DIM, DIM_DST), lambda i: (0, 0)),
        ],
        out_specs=[
            pl.BlockSpec((TBN, DIM), lambda i: (i, 0)),
            pl.BlockSpec((TBN, DIM_DST), lambda i: (i, 0)),
        ],
        out_shape=[
            jax.ShapeDtypeStruct((NP, DIM), jnp.float32),
            jax.ShapeDtypeStruct((NP, DIM_DST), jnp.float32),
        ],
    )(x, m0, m1, wx, w0, w1, bm, wsn)


def kernel(species, edge_src, edge_dst, distances, switch, species_table,
           Ws0, Wm0, bm0, Ws1, Wm1, bm1):
    species = species.astype(jnp.int32)
    src = edge_src.astype(jnp.int32)
    dst = edge_dst.astype(jnp.int32)

    table0 = jnp.dot(species_table, Ws0, preferred_element_type=jnp.float32)
    tcat = jnp.concatenate([species_table, table0], axis=1)
    sp_pad = jnp.zeros((NP,), jnp.int32).at[:N].set(species)

    zs = _sc_gather(tcat, sp_pad)
    s0 = zs[:, ZDIM:]

    m = _sc_edge(s0, src, dst, distances, switch)
    wx0 = jnp.zeros((ZCAT, DIM), jnp.float32).at[:ZDIM].set(Wm0[:ZDIM])
    xi1, s1 = _tc_mix(zs, m[0], m[1], wx0,
                      Wm0[ZDIM:ZDIM + FH], Wm0[ZDIM + FH:ZDIM + 2 * FH],
                      bm0.reshape(1, DIM), Ws1)

    m2 = _sc_edge(s1, src, dst, distances, switch)
    out, _ = _tc_mix(xi1, m2[0], m2[1], Wm1[:DIM],
                     Wm1[DIM:DIM + FH], Wm1[DIM + FH:DIM + 2 * FH],
                     bm1.reshape(1, DIM), Ws1)
    return out[:N]


# speedup vs baseline: 1.2456x; 1.2456x over previous
import functools

import jax
import jax.numpy as jnp
from jax import lax
from jax.experimental import pallas as pl
from jax.experimental.pallas import tpu as pltpu
from jax.experimental.pallas import tpu_sc as plsc

N = 10000
NP = 10240
E = 160000
DIM = 256
DIM_DST = 32
NBASIS = 8
ZDIM = 16
CUTOFF = 5.0
BETA = (NBASIS / CUTOFF) ** 2
CENTERS = [float(i) * CUTOFF / (NBASIS - 1) for i in range(NBASIS)]

NC, NS, L = 2, 16, 16
ZCAT = ZDIM + DIM_DST

EPT = E // NS
CK = 80
NCH = EPT // CK
GPC = CK // L
NB_HALF = NBASIS // NC
FH = NB_HALF * DIM_DST
RPT = NP // NS

_MESH = dict(core_axis_name="c", subcore_axis_name="s", num_cores=NC,
             num_subcores=NS)


def _sc_gather_body(table_hbm, idx_hbm, out_hbm, idxv, rows, sem):
    c = lax.axis_index("c")
    s = lax.axis_index("s")
    wid = s * NC + c
    base = wid * (NP // (NC * NS))
    for j in range(5):
        off = base + j * 64
        pltpu.sync_copy(idx_hbm.at[pl.ds(off, 64)], idxv)
        pltpu.async_copy(table_hbm.at[idxv], rows, sem).wait()
        pltpu.sync_copy(rows, out_hbm.at[pl.ds(off, 64)])


def _sc_gather(table, idx):
    k = pl.kernel(
        _sc_gather_body,
        out_type=jax.ShapeDtypeStruct((NP, ZCAT), jnp.float32),
        mesh=plsc.VectorSubcoreMesh(**_MESH),
        scratch_types=[
            pltpu.VMEM((64,), jnp.int32),
            pltpu.VMEM((64, ZCAT), jnp.float32),
            pltpu.SemaphoreType.DMA,
        ],
        compiler_params=pltpu.CompilerParams(use_tc_tiling_on_sc=False),
    )
    return k(table, idx)


def _sc_edge_body(s_hbm, src_hbm, ed_hbm, out_hbm,
                  srcm, edb, xij, lij, acc, sem_g, sem_s, sem_d):
    c = lax.axis_index("c")
    sid = lax.axis_index("s")
    zero16 = jnp.zeros((L,), jnp.float32)

    pltpu.sync_copy(src_hbm.at[sid], srcm)

    def zrow(r, _):
        for j in range(FH // L):
            lij[0, r, pl.ds(j * L, L)] = zero16
        return 0
    lax.fori_loop(0, CK, zrow, 0)
    for j in range(RPT // CK):
        pltpu.sync_copy(lij.at[0], acc.at[pl.ds(sid * RPT + j * CK, CK)])
    plsc.subcore_barrier()

    def start_edges(i, b):
        pltpu.async_copy(ed_hbm.at[sid, i], edb.at[b], sem_d.at[b])

    def wait_edges(b):
        pltpu.make_async_copy(ed_hbm.at[0, 0], edb.at[b],
                              sem_d.at[b]).wait()

    def start_gather(b):
        pltpu.async_copy(s_hbm.at[edb.at[b, 0]], xij.at[b], sem_g.at[b])

    def wait_gather(b):
        pltpu.make_async_copy(s_hbm.at[pl.ds(0, CK)], xij.at[b],
                              sem_g.at[b]).wait()

    def start_scatter(i, b):
        pltpu.async_copy(lij.at[b], acc.at[srcm.at[i]], sem_s.at[b], add=True)

    def wait_scatter(b):
        pltpu.make_async_copy(lij.at[b], acc.at[pl.ds(0, CK)],
                              sem_s.at[b]).wait()

    def compute(i, b):
        for g in range(GPC):
            dval = plsc.bitcast(edb[b, 1, pl.ds(g * L, L)], jnp.float32)
            swval = plsc.bitcast(edb[b, 2, pl.ds(g * L, L)], jnp.float32)
            rowv = g * L + lax.iota(jnp.int32, L)
            rbs = []
            for bb in range(NB_HALF):
                cb = jnp.where(c == 0, CENTERS[bb], CENTERS[bb + NB_HALF])
                diff = dval - cb
                rbs.append(jnp.exp((-BETA) * diff * diff) * swval)
            for col in range(DIM_DST):
                colv = jnp.full((L,), col, jnp.int32)
                xcol = plsc.load_gather(xij.at[b], [rowv, colv])
                for bb in range(NB_HALF):
                    fcol = jnp.full((L,), bb * DIM_DST + col, jnp.int32)
                    plsc.store_scatter(lij.at[b], [rowv, fcol], rbs[bb] * xcol)

    start_edges(0, 0)
    start_edges(1, 1)
    wait_edges(0)
    start_gather(0)

    def superchunk(j, _):
        for b in range(2):
            i = 2 * j + b
            @pl.when(i + 1 < NCH)
            def _():
                wait_edges(1 - b)
                start_gather(1 - b)
            wait_gather(b)

            @pl.when(j >= 1)
            def _():
                wait_scatter(b)
            compute(i, b)
            start_scatter(i, b)

            @pl.when(i + 2 < NCH)
            def _():
                start_edges(i + 2, b)
        return 0

    lax.fori_loop(0, (NCH - 1) // 2, superchunk, 0)
    wait_gather(0)
    wait_scatter(0)
    compute(NCH - 1, 0)
    start_scatter(NCH - 1, 0)
    wait_scatter(1)
    wait_scatter(0)

    plsc.subcore_barrier()
    r0 = sid * RPT
    pltpu.sync_copy(acc.at[pl.ds(r0, RPT)], out_hbm.at[c, pl.ds(r0, RPT)])


def _sc_edge(s_nodes, src3, ed4):
    k = pl.kernel(
        _sc_edge_body,
        out_type=jax.ShapeDtypeStruct((NC, NP, FH), jnp.float32),
        mesh=plsc.VectorSubcoreMesh(**_MESH),
        scratch_types=[
            pltpu.VMEM((NCH, CK), jnp.int32),
            pltpu.VMEM((2, 3, CK), jnp.int32),
            pltpu.VMEM((2, CK, DIM_DST), jnp.float32),
            pltpu.VMEM((2, CK, FH), jnp.float32),
            pltpu.VMEM_SHARED((NP, FH), jnp.float32),
            pltpu.SemaphoreType.DMA((2,)),
            pltpu.SemaphoreType.DMA((2,)),
            pltpu.SemaphoreType.DMA((2,)),
        ],
        compiler_params=pltpu.CompilerParams(use_tc_tiling_on_sc=False,
                                             needs_layout_passes=False),
    )
    return k(s_nodes, src3, ed4)


TBN = 2048


def _tc_mix_body(x_ref, m0_ref, m1_ref, wx_ref, w0_ref, w1_ref, bm_ref,
                 wsn_ref, xi_ref, s_ref):
    h = jnp.dot(x_ref[...], wx_ref[...], preferred_element_type=jnp.float32)
    h = h + jnp.dot(m0_ref[...], w0_ref[...], preferred_element_type=jnp.float32)
    h = h + jnp.dot(m1_ref[...], w1_ref[...], preferred_element_type=jnp.float32)
    h = h + bm_ref[...]
    xi = h * jnp.exp(jnp.log(1.0 + h * h) * (-1.0 / 3.0))
    xi_ref[...] = xi
    s_ref[...] = jnp.dot(xi, wsn_ref[...], preferred_element_type=jnp.float32)


def _tc_mix(x, m0, m1, wx, w0, w1, bm, wsn):
    kx = x.shape[1]
    return pl.pallas_call(
        _tc_mix_body,
        grid=(NP // TBN,),
        in_specs=[
            pl.BlockSpec((TBN, kx), lambda i: (i, 0)),
            pl.BlockSpec((TBN, FH), lambda i: (i, 0)),
            pl.BlockSpec((TBN, FH), lambda i: (i, 0)),
            pl.BlockSpec((kx, DIM), lambda i: (0, 0)),
            pl.BlockSpec((FH, DIM), lambda i: (0, 0)),
            pl.BlockSpec((FH, DIM), lambda i: (0, 0)),
            pl.BlockSpec((1, DIM), lambda i: (0, 0)),
            pl.BlockSpec((DIM, DIM_DST), lambda i: (0, 0)),
        ],
        out_specs=[
            pl.BlockSpec((TBN, DIM), lambda i: (i, 0)),
            pl.BlockSpec((TBN, DIM_DST), lambda i: (i, 0)),
        ],
        out_shape=[
            jax.ShapeDtypeStruct((NP, DIM), jnp.float32),
            jax.ShapeDtypeStruct((NP, DIM_DST), jnp.float32),
        ],
    )(x, m0, m1, wx, w0, w1, bm, wsn)


def kernel(species, edge_src, edge_dst, distances, switch, species_table,
           Ws0, Wm0, bm0, Ws1, Wm1, bm1):
    species = species.astype(jnp.int32)
    src = edge_src.astype(jnp.int32)
    dst = edge_dst.astype(jnp.int32)

    table0 = jnp.dot(species_table, Ws0, preferred_element_type=jnp.float32)
    tcat = jnp.concatenate([species_table, table0], axis=1)
    sp_pad = jnp.zeros((NP,), jnp.int32).at[:N].set(species)

    src3 = src.reshape(NS, NCH, CK)
    ed4 = jnp.stack([dst.reshape(NS, NCH, CK),
                     lax.bitcast_convert_type(distances, jnp.int32).reshape(NS, NCH, CK),
                     lax.bitcast_convert_type(switch, jnp.int32).reshape(NS, NCH, CK)],
                    axis=2)

    zs = _sc_gather(tcat, sp_pad)
    s0 = zs[:, ZDIM:]

    m = _sc_edge(s0, src3, ed4)
    wx0 = jnp.zeros((ZCAT, DIM), jnp.float32).at[:ZDIM].set(Wm0[:ZDIM])
    xi1, s1 = _tc_mix(zs, m[0], m[1], wx0,
                      Wm0[ZDIM:ZDIM + FH], Wm0[ZDIM + FH:ZDIM + 2 * FH],
                      bm0.reshape(1, DIM), Ws1)

    m2 = _sc_edge(s1, src3, ed4)
    out, _ = _tc_mix(xi1, m2[0], m2[1], Wm1[:DIM],
                     Wm1[DIM:DIM + FH], Wm1[DIM + FH:DIM + 2 * FH],
                     bm1.reshape(1, DIM), Ws1)
    return out[:N]


# speedup vs baseline: 7.0163x; 5.6327x over previous
import functools

import jax
import jax.numpy as jnp
from jax import lax
from jax.experimental import pallas as pl
from jax.experimental.pallas import tpu as pltpu
from jax.experimental.pallas import tpu_sc as plsc

N = 10000
NP = 10240
E = 160000
DIM = 256
DIM_DST = 32
NBASIS = 8
ZDIM = 16
CUTOFF = 5.0
BETA = (NBASIS / CUTOFF) ** 2
CENTERS = [float(i) * CUTOFF / (NBASIS - 1) for i in range(NBASIS)]

NC, NS, L = 2, 16, 16
ZCAT = ZDIM + DIM_DST

EPT = E // NS
CK = 80
NCH = EPT // CK
GPC = CK // L
NB_HALF = NBASIS // NC
FH = NB_HALF * DIM_DST
RPT = NP // NS

_MESH = dict(core_axis_name="c", subcore_axis_name="s", num_cores=NC,
             num_subcores=NS)

_GDN = lax.GatherDimensionNumbers(offset_dims=(), collapsed_slice_dims=(0,),
                                  start_index_map=(0,))


def _bcast_lane(v, e):
    idx = jnp.full((L, 1), e, jnp.int32)
    return lax.gather(v, idx, _GDN, (1,),
                      mode=lax.GatherScatterMode.PROMISE_IN_BOUNDS)


def _sc_gather_body(table_hbm, idx_hbm, out_hbm, idxv, rows, sem):
    c = lax.axis_index("c")
    s = lax.axis_index("s")
    wid = s * NC + c
    base = wid * (NP // (NC * NS))
    for j in range(5):
        off = base + j * 64
        pltpu.sync_copy(idx_hbm.at[pl.ds(off, 64)], idxv)
        pltpu.async_copy(table_hbm.at[idxv], rows, sem).wait()
        pltpu.sync_copy(rows, out_hbm.at[pl.ds(off, 64)])


def _sc_gather(table, idx):
    k = pl.kernel(
        _sc_gather_body,
        out_type=jax.ShapeDtypeStruct((NP, ZCAT), jnp.float32),
        mesh=plsc.VectorSubcoreMesh(**_MESH),
        scratch_types=[
            pltpu.VMEM((64,), jnp.int32),
            pltpu.VMEM((64, ZCAT), jnp.float32),
            pltpu.SemaphoreType.DMA,
        ],
        compiler_params=pltpu.CompilerParams(use_tc_tiling_on_sc=False),
    )
    return k(table, idx)


def _sc_edge_body(s_hbm, src_hbm, ed_hbm, out_hbm,
                  srcm, edb, xij, lij, acc, sem_g, sem_s, sem_d):
    c = lax.axis_index("c")
    sid = lax.axis_index("s")
    zero16 = jnp.zeros((L,), jnp.float32)

    pltpu.sync_copy(src_hbm.at[sid], srcm)

    def zrow(r, _):
        for j in range(FH // L):
            lij[0, r, pl.ds(j * L, L)] = zero16
        return 0
    lax.fori_loop(0, CK, zrow, 0)
    for j in range(RPT // CK):
        pltpu.sync_copy(lij.at[0], acc.at[pl.ds(sid * RPT + j * CK, CK)])
    plsc.subcore_barrier()

    def start_edges(i, b):
        pltpu.async_copy(ed_hbm.at[sid, i], edb.at[b], sem_d.at[b])

    def wait_edges(b):
        pltpu.make_async_copy(ed_hbm.at[0, 0], edb.at[b],
                              sem_d.at[b]).wait()

    def start_gather(b):
        pltpu.async_copy(s_hbm.at[edb.at[b, 0]], xij.at[b], sem_g.at[b])

    def wait_gather(b):
        pltpu.make_async_copy(s_hbm.at[pl.ds(0, CK)], xij.at[b],
                              sem_g.at[b]).wait()

    def start_scatter(i, b):
        pltpu.async_copy(lij.at[b], acc.at[srcm.at[i]], sem_s.at[b], add=True)

    def wait_scatter(b):
        pltpu.make_async_copy(lij.at[b], acc.at[pl.ds(0, CK)],
                              sem_s.at[b]).wait()

    def compute(i, b):
        for g in range(GPC):
            dval = plsc.bitcast(edb[b, 1, pl.ds(g * L, L)], jnp.float32)
            swval = plsc.bitcast(edb[b, 2, pl.ds(g * L, L)], jnp.float32)
            rbs = []
            for bb in range(NB_HALF):
                cb = jnp.where(c == 0, CENTERS[bb], CENTERS[bb + NB_HALF])
                diff = dval - cb
                rbs.append(jnp.exp((-BETA) * diff * diff) * swval)
            for e in range(L):
                row = g * L + e
                x0 = xij[b, row, pl.ds(0, L)]
                x1 = xij[b, row, pl.ds(L, L)]
                for bb in range(NB_HALF):
                    rbe = _bcast_lane(rbs[bb], e)
                    f0 = bb * DIM_DST
                    lij[b, row, pl.ds(f0, L)] = rbe * x0
                    lij[b, row, pl.ds(f0 + L, L)] = rbe * x1

    start_edges(0, 0)
    start_edges(1, 1)
    wait_edges(0)
    start_gather(0)

    def superchunk(j, _):
        for b in range(2):
            i = 2 * j + b
            @pl.when(i + 1 < NCH)
            def _():
                wait_edges(1 - b)
                start_gather(1 - b)
            wait_gather(b)

            @pl.when(j >= 1)
            def _():
                wait_scatter(b)
            compute(i, b)
            start_scatter(i, b)

            @pl.when(i + 2 < NCH)
            def _():
                start_edges(i + 2, b)
        return 0

    lax.fori_loop(0, (NCH - 1) // 2, superchunk, 0)
    wait_gather(0)
    wait_scatter(0)
    compute(NCH - 1, 0)
    start_scatter(NCH - 1, 0)
    wait_scatter(1)
    wait_scatter(0)

    plsc.subcore_barrier()
    r0 = sid * RPT
    pltpu.sync_copy(acc.at[pl.ds(r0, RPT)], out_hbm.at[c, pl.ds(r0, RPT)])


def _sc_edge(s_nodes, src3, ed4):
    k = pl.kernel(
        _sc_edge_body,
        out_type=jax.ShapeDtypeStruct((NC, NP, FH), jnp.float32),
        mesh=plsc.VectorSubcoreMesh(**_MESH),
        scratch_types=[
            pltpu.VMEM((NCH, CK), jnp.int32),
            pltpu.VMEM((2, 3, CK), jnp.int32),
            pltpu.VMEM((2, CK, DIM_DST), jnp.float32),
            pltpu.VMEM((2, CK, FH), jnp.float32),
            pltpu.VMEM_SHARED((NP, FH), jnp.float32),
            pltpu.SemaphoreType.DMA((2,)),
            pltpu.SemaphoreType.DMA((2,)),
            pltpu.SemaphoreType.DMA((2,)),
        ],
        compiler_params=pltpu.CompilerParams(use_tc_tiling_on_sc=False,
                                             needs_layout_passes=False),
    )
    return k(s_nodes, src3, ed4)


TBN = 2048


def _tc_mix_body(x_ref, m0_ref, m1_ref, wx_ref, w0_ref, w1_ref, bm_ref,
                 wsn_ref, xi_ref, s_ref):
    h = jnp.dot(x_ref[...], wx_ref[...], preferred_element_type=jnp.float32)
    h = h + jnp.dot(m0_ref[...], w0_ref[...], preferred_element_type=jnp.float32)
    h = h + jnp.dot(m1_ref[...], w1_ref[...], preferred_element_type=jnp.float32)
    h = h + bm_ref[...]
    xi = h * jnp.exp(jnp.log(1.0 + h * h) * (-1.0 / 3.0))
    xi_ref[...] = xi
    s_ref[...] = jnp.dot(xi, wsn_ref[...], preferred_element_type=jnp.float32)


def _tc_mix(x, m0, m1, wx, w0, w1, bm, wsn):
    kx = x.shape[1]
    return pl.pallas_call(
        _tc_mix_body,
        grid=(NP // TBN,),
        in_specs=[
            pl.BlockSpec((TBN, kx), lambda i: (i, 0)),
            pl.BlockSpec((TBN, FH), lambda i: (i, 0)),
            pl.BlockSpec((TBN, FH), lambda i: (i, 0)),
            pl.BlockSpec((kx, DIM), lambda i: (0, 0)),
            pl.BlockSpec((FH, DIM), lambda i: (0, 0)),
            pl.BlockSpec((FH, DIM), lambda i: (0, 0)),
            pl.BlockSpec((1, DIM), lambda i: (0, 0)),
            pl.BlockSpec((DIM, DIM_DST), lambda i: (0, 0)),
        ],
        out_specs=[
            pl.BlockSpec((TBN, DIM), lambda i: (i, 0)),
            pl.BlockSpec((TBN, DIM_DST), lambda i: (i, 0)),
        ],
        out_shape=[
            jax.ShapeDtypeStruct((NP, DIM), jnp.float32),
            jax.ShapeDtypeStruct((NP, DIM_DST), jnp.float32),
        ],
    )(x, m0, m1, wx, w0, w1, bm, wsn)


def kernel(species, edge_src, edge_dst, distances, switch, species_table,
           Ws0, Wm0, bm0, Ws1, Wm1, bm1):
    species = species.astype(jnp.int32)
    src = edge_src.astype(jnp.int32)
    dst = edge_dst.astype(jnp.int32)

    table0 = jnp.dot(species_table, Ws0, preferred_element_type=jnp.float32)
    tcat = jnp.concatenate([species_table, table0], axis=1)
    sp_pad = jnp.zeros((NP,), jnp.int32).at[:N].set(species)

    src3 = src.reshape(NS, NCH, CK)
    ed4 = jnp.stack([dst.reshape(NS, NCH, CK),
                     lax.bitcast_convert_type(distances, jnp.int32).reshape(NS, NCH, CK),
                     lax.bitcast_convert_type(switch, jnp.int32).reshape(NS, NCH, CK)],
                    axis=2)

    zs = _sc_gather(tcat, sp_pad)
    s0 = zs[:, ZDIM:]

    m = _sc_edge(s0, src3, ed4)
    wx0 = jnp.zeros((ZCAT, DIM), jnp.float32).at[:ZDIM].set(Wm0[:ZDIM])
    xi1, s1 = _tc_mix(zs, m[0], m[1], wx0,
                      Wm0[ZDIM:ZDIM + FH], Wm0[ZDIM + FH:ZDIM + 2 * FH],
                      bm0.reshape(1, DIM), Ws1)

    m2 = _sc_edge(s1, src3, ed4)
    out, _ = _tc_mix(xi1, m2[0], m2[1], Wm1[:DIM],
                     Wm1[DIM:DIM + FH], Wm1[DIM + FH:DIM + 2 * FH],
                     bm1.reshape(1, DIM), Ws1)
    return out[:N]
